# baseline (speedup 1.0000x reference)
import functools

import jax
import jax.numpy as jnp
from jax.experimental import pallas as pl
from jax.experimental.pallas import tpu as pltpu
from jax.experimental.pallas import tpu_sc as plsc

N = 32768
D = 256
H = 8
B = 16
BH = B * H
BLK = 8192

NC = 2
NS = 16
NW = NC * NS
RPW = N // NW
STEPS = RPW // 16


def _agg_kernel(seg_ref, att_ref, feat0_ref, feat1_ref, sum_ref, w_ref):
    i = pl.program_id(0)
    nsteps = pl.num_programs(0)

    @pl.when(i == 0)
    def _init():
        sum_ref[...] = jnp.zeros_like(sum_ref)
        w_ref[...] = jnp.zeros_like(w_ref)

    seg = seg_ref[...]
    att_t = att_ref[...]
    feat0 = feat0_ref[...]
    feat1 = feat1_ref[...]

    b_idx = jax.lax.broadcasted_iota(jnp.int32, (BH, BLK), 0) // H
    seg_b = jnp.broadcast_to(seg, (BH, BLK))
    att_rep = jnp.tile(att_t, (B, 1))
    pt = jnp.where(seg_b == b_idx, att_rep, 0.0)

    dn = (((1,), (0,)), ((), ()))
    sum_ref[:, : D // 2] += jax.lax.dot_general(
        pt, feat0, dn, preferred_element_type=jnp.float32
    )
    sum_ref[:, D // 2 :] += jax.lax.dot_general(
        pt, feat1, dn, preferred_element_type=jnp.float32
    )
    w_ref[...] += jax.lax.dot_general(
        pt, jnp.ones((BLK, 1), jnp.float32), dn,
        preferred_element_type=jnp.float32,
    )

    @pl.when(i == nsteps - 1)
    def _finalize():
        w = w_ref[...]
        safe = jnp.where(w == 0.0, 1.0, w)
        avg = jnp.where(w == 0.0, 0.0, sum_ref[...] / safe)
        avg = jnp.where(jnp.isnan(avg), 1e-05, avg)
        sum_ref[...] = avg


def _sc_weights_body(att_hbm, seg_hbm, out_hbm, att_v, seg_v, acc_v, part_v,
                     shared):
    cid = jax.lax.axis_index("c")
    sid = jax.lax.axis_index("s")
    wid = sid * NC + cid
    base = wid * RPW

    pltpu.sync_copy(att_hbm.at[:, pl.ds(base, RPW)], att_v)
    pltpu.sync_copy(seg_hbm.at[pl.ds(base, RPW)], seg_v)

    lane = jax.lax.broadcasted_iota(jnp.int32, (16,), 0)
    lane_bank = lane * BH

    zeros = jnp.zeros((16,), jnp.float32)
    for k in range(16 * BH // 16):
        acc_v[pl.ds(k * 16, 16)] = zeros

    def step(j, carry):
        segs = seg_v[pl.ds(j * 16, 16)]
        idx0 = lane_bank + segs * H
        for h in range(H):
            plsc.addupdate_scatter(
                acc_v, [idx0 + h], att_v[h, pl.ds(j * 16, 16)]
            )
        return carry

    jax.lax.fori_loop(0, STEPS, step, 0)

    for k in range(BH // 16):
        s = acc_v[pl.ds(k * 16, 16)]
        for r in range(1, 16):
            s = s + acc_v[pl.ds(r * BH + k * 16, 16)]
        acc_v[pl.ds(k * 16, 16)] = s
    pltpu.sync_copy(acc_v.at[pl.ds(0, BH)], shared.at[sid])
    plsc.subcore_barrier()

    @pl.when(sid == 0)
    def _combine():
        pltpu.sync_copy(shared, part_v)
        for k in range(BH // 16):
            s = part_v[0, pl.ds(k * 16, 16)]
            for r in range(1, NS):
                s = s + part_v[r, pl.ds(k * 16, 16)]
            acc_v[pl.ds(k * 16, 16)] = s
        pltpu.sync_copy(acc_v.at[pl.ds(0, BH)], out_hbm.at[cid])


_sc_weights = functools.partial(
    pl.kernel,
    out_type=jax.ShapeDtypeStruct((NC, BH), jnp.float32),
    mesh=plsc.VectorSubcoreMesh(core_axis_name="c", subcore_axis_name="s"),
    compiler_params=pltpu.CompilerParams(
        needs_layout_passes=False, skip_device_barrier=True
    ),
    scratch_types=[
        pltpu.VMEM((H, RPW), jnp.float32),
        pltpu.VMEM((RPW,), jnp.int32),
        pltpu.VMEM((16 * BH,), jnp.float32),
        pltpu.VMEM((NS, BH), jnp.float32),
        pltpu.VMEM_SHARED((NS, BH), jnp.float32),
    ],
)(_sc_weights_body)


def kernel(flat_features, flat_att, segment_ids):
    seg2d = segment_ids.reshape(1, N)
    att_t = flat_att.T
    grid = N // BLK
    avg, w = pl.pallas_call(
        _agg_kernel,
        grid=(grid,),
        in_specs=[
            pl.BlockSpec((1, BLK), lambda i: (0, i)),
            pl.BlockSpec((H, BLK), lambda i: (0, i)),
            pl.BlockSpec((BLK, D // 2), lambda i: (i, 0)),
            pl.BlockSpec((BLK, D // 2), lambda i: (i, 1)),
        ],
        out_specs=[
            pl.BlockSpec((BH, D), lambda i: (0, 0)),
            pl.BlockSpec((BH, 1), lambda i: (0, 0)),
        ],
        out_shape=[
            jax.ShapeDtypeStruct((BH, D), jnp.float32),
            jax.ShapeDtypeStruct((BH, 1), jnp.float32),
        ],
    )(seg2d, att_t, flat_features, flat_features)
    return avg.reshape(B, H, D), w.reshape(B, H)


# speedup vs baseline: 1.0749x; 1.0749x over previous
import functools

import jax
import jax.numpy as jnp
from jax.experimental import pallas as pl
from jax.experimental.pallas import tpu as pltpu
from jax.experimental.pallas import tpu_sc as plsc

N = 32768
D = 256
H = 8
B = 16
BH = B * H
BLK = 8192

NC = 2
NS = 16
NW = NC * NS
RPW = N // NW
STEPS = RPW // 16


def _agg_kernel(seg_ref, att_ref, feat_ref, sum_ref, w_ref):
    i = pl.program_id(0)
    nsteps = pl.num_programs(0)

    @pl.when(i == 0)
    def _init():
        sum_ref[...] = jnp.zeros_like(sum_ref)
        w_ref[...] = jnp.zeros_like(w_ref)

    seg = seg_ref[...]
    att_t = att_ref[...]
    feat = feat_ref[...]

    b_idx = jax.lax.broadcasted_iota(jnp.int32, (BH, BLK), 0) // H
    seg_b = jnp.broadcast_to(seg, (BH, BLK))
    att_rep = jnp.tile(att_t, (B, 1))
    pt = jnp.where(seg_b == b_idx, att_rep, 0.0)

    dn = (((1,), (0,)), ((), ()))
    sum_ref[...] += jax.lax.dot_general(
        pt, feat, dn, preferred_element_type=jnp.float32
    )
    w_ref[...] += jax.lax.dot_general(
        pt, jnp.ones((BLK, 1), jnp.float32), dn,
        preferred_element_type=jnp.float32,
    )

    @pl.when(i == nsteps - 1)
    def _finalize():
        w = w_ref[...]
        safe = jnp.where(w == 0.0, 1.0, w)
        avg = jnp.where(w == 0.0, 0.0, sum_ref[...] / safe)
        avg = jnp.where(jnp.isnan(avg), 1e-05, avg)
        sum_ref[...] = avg


def _sc_weights_body(att_hbm, seg_hbm, out_hbm, att_v, seg_v, acc_v, part_v,
                     shared):
    cid = jax.lax.axis_index("c")
    sid = jax.lax.axis_index("s")
    wid = sid * NC + cid
    base = wid * RPW

    pltpu.sync_copy(att_hbm.at[:, pl.ds(base, RPW)], att_v)
    pltpu.sync_copy(seg_hbm.at[pl.ds(base, RPW)], seg_v)

    lane = jax.lax.broadcasted_iota(jnp.int32, (16,), 0)
    lane_bank = lane * BH

    zeros = jnp.zeros((16,), jnp.float32)
    for k in range(16 * BH // 16):
        acc_v[pl.ds(k * 16, 16)] = zeros

    def step(j, carry):
        segs = seg_v[pl.ds(j * 16, 16)]
        idx0 = lane_bank + segs * H
        for h in range(H):
            plsc.addupdate_scatter(
                acc_v, [idx0 + h], att_v[h, pl.ds(j * 16, 16)]
            )
        return carry

    jax.lax.fori_loop(0, STEPS, step, 0)

    for k in range(BH // 16):
        s = acc_v[pl.ds(k * 16, 16)]
        for r in range(1, 16):
            s = s + acc_v[pl.ds(r * BH + k * 16, 16)]
        acc_v[pl.ds(k * 16, 16)] = s
    pltpu.sync_copy(acc_v.at[pl.ds(0, BH)], shared.at[sid])
    plsc.subcore_barrier()

    @pl.when(sid == 0)
    def _combine():
        pltpu.sync_copy(shared, part_v)
        for k in range(BH // 16):
            s = part_v[0, pl.ds(k * 16, 16)]
            for r in range(1, NS):
                s = s + part_v[r, pl.ds(k * 16, 16)]
            acc_v[pl.ds(k * 16, 16)] = s
        pltpu.sync_copy(acc_v.at[pl.ds(0, BH)], out_hbm.at[cid])


_sc_weights = functools.partial(
    pl.kernel,
    out_type=jax.ShapeDtypeStruct((NC, BH), jnp.float32),
    mesh=plsc.VectorSubcoreMesh(core_axis_name="c", subcore_axis_name="s"),
    compiler_params=pltpu.CompilerParams(
        needs_layout_passes=False, skip_device_barrier=True
    ),
    scratch_types=[
        pltpu.VMEM((H, RPW), jnp.float32),
        pltpu.VMEM((RPW,), jnp.int32),
        pltpu.VMEM((16 * BH,), jnp.float32),
        pltpu.VMEM((NS, BH), jnp.float32),
        pltpu.VMEM_SHARED((NS, BH), jnp.float32),
    ],
)(_sc_weights_body)


def kernel(flat_features, flat_att, segment_ids):
    seg2d = segment_ids.reshape(1, N)
    att_t = flat_att.T
    grid = N // BLK
    avg, w = pl.pallas_call(
        _agg_kernel,
        grid=(grid,),
        in_specs=[
            pl.BlockSpec((1, BLK), lambda i: (0, i)),
            pl.BlockSpec((H, BLK), lambda i: (0, i)),
            pl.BlockSpec((BLK, D), lambda i: (i, 0)),
        ],
        out_specs=[
            pl.BlockSpec((BH, D), lambda i: (0, 0)),
            pl.BlockSpec((BH, 1), lambda i: (0, 0)),
        ],
        out_shape=[
            jax.ShapeDtypeStruct((BH, D), jnp.float32),
            jax.ShapeDtypeStruct((BH, 1), jnp.float32),
        ],
    )(seg2d, att_t, flat_features)
    return avg.reshape(B, H, D), w.reshape(B, H)


# speedup vs baseline: 1.1007x; 1.0240x over previous
import jax
import jax.numpy as jnp
from jax.experimental import pallas as pl
from jax.experimental.pallas import tpu as pltpu

N = 32768
D = 256
H = 8
B = 16
BH = B * H
BLK = 8192


def _agg_kernel(seg_ref, att_ref, feat_ref, avg_ref, w_ref, sum_acc, w_acc):
    i = pl.program_id(0)
    nsteps = pl.num_programs(0)

    @pl.when(i == 0)
    def _init():
        sum_acc[...] = jnp.zeros_like(sum_acc)
        w_acc[...] = jnp.zeros_like(w_acc)

    seg = seg_ref[...]
    att_t = att_ref[...]
    feat = feat_ref[...]

    b_idx = jax.lax.broadcasted_iota(jnp.int32, (BH, BLK), 0) // H
    seg_b = jnp.broadcast_to(seg, (BH, BLK))
    att_rep = jnp.tile(att_t, (B, 1))
    pt = jnp.where(seg_b == b_idx, att_rep, 0.0)

    dn = (((1,), (0,)), ((), ()))
    sum_acc[...] += jax.lax.dot_general(
        pt, feat, dn, preferred_element_type=jnp.float32
    )
    w_acc[...] += jax.lax.dot_general(
        pt, jnp.ones((BLK, 1), jnp.float32), dn,
        preferred_element_type=jnp.float32,
    )

    @pl.when(i == nsteps - 1)
    def _finalize():
        w = w_acc[...]
        safe = jnp.where(w == 0.0, 1.0, w)
        avg = jnp.where(w == 0.0, 0.0, sum_acc[...] / safe)
        avg = jnp.where(jnp.isnan(avg), 1e-05, avg)
        avg_ref[...] = avg.reshape(B, H, D)
        w_ref[...] = w.reshape(B, H)


def kernel(flat_features, flat_att, segment_ids):
    seg2d = segment_ids.reshape(1, N)
    att_t = flat_att.T
    grid = N // BLK
    avg, w = pl.pallas_call(
        _agg_kernel,
        grid=(grid,),
        in_specs=[
            pl.BlockSpec((1, BLK), lambda i: (0, i)),
            pl.BlockSpec((H, BLK), lambda i: (0, i)),
            pl.BlockSpec((BLK, D), lambda i: (i, 0)),
        ],
        out_specs=[
            pl.BlockSpec((B, H, D), lambda i: (0, 0, 0)),
            pl.BlockSpec((B, H), lambda i: (0, 0)),
        ],
        out_shape=[
            jax.ShapeDtypeStruct((B, H, D), jnp.float32),
            jax.ShapeDtypeStruct((B, H), jnp.float32),
        ],
        scratch_shapes=[
            pltpu.VMEM((BH, D), jnp.float32),
            pltpu.VMEM((BH, 1), jnp.float32),
        ],
    )(seg2d, att_t, flat_features)
    return avg, w
